# Initial kernel scaffold; baseline (speedup 1.0000x reference)
import jax
import jax.numpy as jnp
from jax.experimental import pallas as pl


def kernel(x, Wr, br, W1, b1, W2, b2):
    raise NotImplementedError("write your pallas kernel here")


# speedup vs baseline: 1.0862x; 1.0862x over previous
import jax
import jax.numpy as jnp
from jax.experimental import pallas as pl
from jax.experimental.pallas import tpu as pltpu

B, N, D_MODEL, D_FF, E = 4, 8192, 768, 1024, 8

_TR = 1024
_TN = 1024


def _router_body(x_ref, wr_ref, br_ref, idx_ref, acc_ref):
    b = pl.program_id(0)
    n = pl.program_id(1)
    nt = pl.num_programs(1)

    part = jnp.sum(x_ref[0], axis=0, keepdims=True)

    @pl.when(n == 0)
    def _init():
        acc_ref[pl.ds(b, 1), :] = part

    @pl.when(n != 0)
    def _acc():
        acc_ref[pl.ds(b, 1), :] = acc_ref[pl.ds(b, 1), :] + part

    @pl.when((b == B - 1) & (n == nt - 1))
    def _final():
        pooled = acc_ref[...] * (1.0 / N)
        scores = jax.lax.dot_general(
            pooled, wr_ref[...],
            (((1,), (1,)), ((), ())),
            preferred_element_type=jnp.float32,
        ) + br_ref[...]
        top1 = jnp.argmax(scores, axis=-1).astype(jnp.int32)
        idx_ref[...] = jnp.broadcast_to(top1[:, None], (B, E))


def _ffn_body(top1_ref, x_ref, w1_ref, b1_ref, w2_ref, b2_ref, out_ref):
    x = x_ref[0]
    h = jnp.dot(x, w1_ref[0], preferred_element_type=jnp.float32)
    h = jnp.maximum(h + b1_ref[0], 0.0)
    out = jnp.dot(h, w2_ref[0], preferred_element_type=jnp.float32)
    out_ref[0] = out + b2_ref[0]


def kernel(x, Wr, br, W1, b1, W2, b2):
    idx = pl.pallas_call(
        _router_body,
        grid=(B, N // _TR),
        in_specs=[
            pl.BlockSpec((1, _TR, D_MODEL), lambda b, n: (b, n, 0)),
            pl.BlockSpec((E, D_MODEL), lambda b, n: (0, 0)),
            pl.BlockSpec((1, E), lambda b, n: (0, 0)),
        ],
        out_specs=pl.BlockSpec((B, E), lambda b, n: (0, 0)),
        out_shape=jax.ShapeDtypeStruct((B, E), jnp.int32),
        scratch_shapes=[pltpu.VMEM((B, D_MODEL), jnp.float32)],
    )(x, Wr, br.reshape(1, E))
    top1 = idx[:, 0]

    grid_spec = pltpu.PrefetchScalarGridSpec(
        num_scalar_prefetch=1,
        grid=(B, N // _TN),
        in_specs=[
            pl.BlockSpec((1, _TN, D_MODEL), lambda b, n, t: (b, n, 0)),
            pl.BlockSpec((1, D_MODEL, D_FF), lambda b, n, t: (t[b], 0, 0)),
            pl.BlockSpec((1, 1, D_FF), lambda b, n, t: (t[b], 0, 0)),
            pl.BlockSpec((1, D_FF, D_MODEL), lambda b, n, t: (t[b], 0, 0)),
            pl.BlockSpec((1, 1, D_MODEL), lambda b, n, t: (t[b], 0, 0)),
        ],
        out_specs=pl.BlockSpec((1, _TN, D_MODEL), lambda b, n, t: (b, n, 0)),
    )
    out = pl.pallas_call(
        _ffn_body,
        grid_spec=grid_spec,
        out_shape=jax.ShapeDtypeStruct((B, N, D_MODEL), jnp.float32),
    )(top1, x, W1, b1.reshape(E, 1, D_FF), W2, b2.reshape(E, 1, D_MODEL))
    return out
